# baseline (speedup 1.0000x reference)
import jax
import jax.numpy as jnp
from jax.experimental import pallas as pl
from jax.experimental.pallas import tpu as pltpu

_N = 10000
_IN_F = 128
_MID = 32
_OUT_F = 128
_BM = 400
_NBUF = 3
_NSTEPS = _N // _BM


def _gcn_kernel(x_ref, w1_ref, w2_ref, adj_ref, out_ref, abuf, hid_ref, sems):
    i = pl.program_id(0)

    @pl.when(i == 0)
    def _():
        for b in range(_NBUF):
            pltpu.make_async_copy(
                adj_ref.at[pl.ds(b * _BM, _BM), :], abuf.at[b], sems.at[b]
            ).start()
        h = jnp.dot(
            x_ref[...].astype(jnp.bfloat16),
            w1_ref[...].astype(jnp.bfloat16),
            preferred_element_type=jnp.float32,
        )
        hid_ref[...] = h.astype(jnp.bfloat16)

    slot = jax.lax.rem(i, _NBUF)
    pltpu.make_async_copy(
        adj_ref.at[pl.ds(i * _BM, _BM), :], abuf.at[slot], sems.at[slot]
    ).wait()
    t = jnp.dot(
        abuf[slot].astype(jnp.bfloat16),
        hid_ref[...],
        preferred_element_type=jnp.float32,
    )
    out_ref[...] = jnp.dot(
        t.astype(jnp.bfloat16),
        w2_ref[...].astype(jnp.bfloat16),
        preferred_element_type=jnp.float32,
    )

    @pl.when(i + _NBUF < _NSTEPS)
    def _():
        pltpu.make_async_copy(
            adj_ref.at[pl.ds((i + _NBUF) * _BM, _BM), :], abuf.at[slot],
            sems.at[slot],
        ).start()


def kernel(input, adj, weight1, weight2):
    return pl.pallas_call(
        _gcn_kernel,
        grid=(_NSTEPS,),
        in_specs=[
            pl.BlockSpec((_N, _IN_F), lambda i: (0, 0)),
            pl.BlockSpec((_IN_F, _MID), lambda i: (0, 0)),
            pl.BlockSpec((_MID, _OUT_F), lambda i: (0, 0)),
            pl.BlockSpec(memory_space=pltpu.MemorySpace.HBM),
        ],
        out_specs=pl.BlockSpec((_BM, _OUT_F), lambda i: (i, 0)),
        out_shape=jax.ShapeDtypeStruct((_N, _OUT_F), jnp.float32),
        scratch_shapes=[
            pltpu.VMEM((_NBUF, _BM, _N), jnp.float32),
            pltpu.VMEM((_N, _MID), jnp.bfloat16),
            pltpu.SemaphoreType.DMA((_NBUF,)),
        ],
        compiler_params=pltpu.CompilerParams(
            dimension_semantics=("arbitrary",),
        ),
    )(input, weight1, weight2, adj)


# speedup vs baseline: 1.0335x; 1.0335x over previous
import jax
import jax.numpy as jnp
from jax.experimental import pallas as pl
from jax.experimental.pallas import tpu as pltpu

_N = 10000
_IN_F = 128
_MID = 32
_OUT_F = 128
_BM = 400


def _gcn_kernel(x_ref, w1_ref, adj_ref, w2_ref, out_ref, hid_ref):
    @pl.when(pl.program_id(0) == 0)
    def _():
        h = jnp.dot(
            x_ref[...].astype(jnp.bfloat16),
            w1_ref[...].astype(jnp.bfloat16),
            preferred_element_type=jnp.float32,
        )
        hid_ref[...] = h.astype(jnp.bfloat16)

    t = jnp.dot(
        adj_ref[...].astype(jnp.bfloat16),
        hid_ref[...],
        preferred_element_type=jnp.float32,
    )
    out_ref[...] = jnp.dot(
        t.astype(jnp.bfloat16),
        w2_ref[...].astype(jnp.bfloat16),
        preferred_element_type=jnp.float32,
    )


def kernel(input, adj, weight1, weight2):
    grid = (_N // _BM,)
    return pl.pallas_call(
        _gcn_kernel,
        grid=grid,
        in_specs=[
            pl.BlockSpec((_N, _IN_F), lambda i: (0, 0)),
            pl.BlockSpec((_IN_F, _MID), lambda i: (0, 0)),
            pl.BlockSpec((_BM, _N), lambda i: (i, 0)),
            pl.BlockSpec((_MID, _OUT_F), lambda i: (0, 0)),
        ],
        out_specs=pl.BlockSpec((_BM, _OUT_F), lambda i: (i, 0)),
        out_shape=jax.ShapeDtypeStruct((_N, _OUT_F), jnp.float32),
        scratch_shapes=[pltpu.VMEM((_N, _MID), jnp.bfloat16)],
        compiler_params=pltpu.CompilerParams(
            dimension_semantics=("arbitrary",),
        ),
    )(input, weight1, adj, weight2)
